# Initial kernel scaffold; baseline (speedup 1.0000x reference)
import jax
import jax.numpy as jnp
from jax.experimental import pallas as pl


def kernel(char_ids, char_embs):
    raise NotImplementedError("write your pallas kernel here")


# speedup vs baseline: 3.9344x; 3.9344x over previous
import functools

import jax
import jax.numpy as jnp
from jax import lax
from jax.experimental import pallas as pl
from jax.experimental.pallas import tpu as pltpu
from jax.experimental.pallas import tpu_sc as plsc

_VOCAB = 256
_EMB = 32
_B = 4096
_L = 200
_N = _B * _L
_NC = 2
_NS = 16
_NW = _NC * _NS
_N_PER_W = _N // _NW
_CHUNK = 1024
_N_CHUNKS = _N_PER_W // _CHUNK

_mesh = plsc.VectorSubcoreMesh(core_axis_name="c", subcore_axis_name="s")


@functools.partial(
    pl.kernel,
    out_type=jax.ShapeDtypeStruct((_N, _EMB), jnp.float32),
    mesh=_mesh,
    scratch_types=[
        pltpu.VMEM((_CHUNK,), jnp.int32),
        pltpu.VMEM((_CHUNK, _EMB), jnp.float32),
        pltpu.SemaphoreType.DMA,
    ],
    compiler_params=pltpu.CompilerParams(use_tc_tiling_on_sc=False),
)
def _gather_kernel(ids_hbm, table_hbm, out_hbm, idx_v, rows_v, sem):
    wid = lax.axis_index("s") * _NC + lax.axis_index("c")
    base = wid * _N_PER_W

    @pl.loop(0, _N_CHUNKS)
    def _(i):
        off = base + i * _CHUNK
        pltpu.sync_copy(ids_hbm.at[pl.ds(off, _CHUNK)], idx_v)
        pltpu.async_copy(table_hbm.at[idx_v], rows_v, sem).wait()
        pltpu.sync_copy(rows_v, out_hbm.at[pl.ds(off, _CHUNK)])


def kernel(char_ids, char_embs):
    ids_flat = char_ids.reshape(_N)
    out = _gather_kernel(ids_flat, char_embs)
    return out.reshape(_B, _L, _EMB)


# speedup vs baseline: 6.0284x; 1.5322x over previous
import functools

import jax
import jax.numpy as jnp
from jax import lax
from jax.experimental import pallas as pl
from jax.experimental.pallas import tpu as pltpu
from jax.experimental.pallas import tpu_sc as plsc

_VOCAB = 256
_EMB = 32
_B = 4096
_L = 200
_N = _B * _L
_NC = 2
_NS = 16
_NW = _NC * _NS
_N_PER_W = _N // _NW
_CHUNK = 1600
_N_CHUNKS = _N_PER_W // _CHUNK

_mesh = plsc.VectorSubcoreMesh(core_axis_name="c", subcore_axis_name="s")


@functools.partial(
    pl.kernel,
    out_type=jax.ShapeDtypeStruct((_N, _EMB), jnp.float32),
    mesh=_mesh,
    scratch_types=[
        pltpu.VMEM_SHARED((_VOCAB, _EMB), jnp.float32),
        pltpu.VMEM((2, _CHUNK), jnp.int32),
        pltpu.VMEM((2, _CHUNK, _EMB), jnp.float32),
        pltpu.SemaphoreType.DMA((2,)),
        pltpu.SemaphoreType.DMA((2,)),
        pltpu.SemaphoreType.DMA((2,)),
    ],
    compiler_params=pltpu.CompilerParams(use_tc_tiling_on_sc=False),
)
def _gather_kernel(ids_hbm, table_hbm, out_hbm, table_v, idx_v, rows_v,
                   sem_idx, sem_g, sem_w):
    wid = lax.axis_index("s") * _NC + lax.axis_index("c")
    base = wid * _N_PER_W

    @pl.when(lax.axis_index("s") == 0)
    def _():
        pltpu.sync_copy(table_hbm, table_v)
    plsc.subcore_barrier()

    for s in range(2):
        pltpu.async_copy(ids_hbm.at[pl.ds(base + s * _CHUNK, _CHUNK)],
                         idx_v.at[s], sem_idx.at[s])

    @pl.loop(0, _N_CHUNKS, step=2)
    def _steady(i):
        for s in range(2):
            c = i + s
            off = base + c * _CHUNK
            pltpu.make_async_copy(ids_hbm.at[pl.ds(off, _CHUNK)],
                                  idx_v.at[s], sem_idx.at[s]).wait()

            @pl.when(c >= 2)
            def _():
                pltpu.make_async_copy(
                    rows_v.at[s],
                    out_hbm.at[pl.ds(off - 2 * _CHUNK, _CHUNK)],
                    sem_w.at[s]).wait()

            pltpu.async_copy(table_v.at[idx_v.at[s]], rows_v.at[s], sem_g.at[s])
            pltpu.make_async_copy(table_v.at[idx_v.at[s]], rows_v.at[s],
                                  sem_g.at[s]).wait()

            pltpu.async_copy(rows_v.at[s], out_hbm.at[pl.ds(off, _CHUNK)],
                             sem_w.at[s])

            @pl.when(c + 2 < _N_CHUNKS)
            def _():
                pltpu.async_copy(ids_hbm.at[pl.ds(off + 2 * _CHUNK, _CHUNK)],
                                 idx_v.at[s], sem_idx.at[s])

    for s in range(2):
        off = base + (_N_CHUNKS - 2 + s) * _CHUNK
        pltpu.make_async_copy(rows_v.at[s], out_hbm.at[pl.ds(off, _CHUNK)],
                              sem_w.at[s]).wait()


def kernel(char_ids, char_embs):
    ids_flat = char_ids.reshape(_N)
    out = _gather_kernel(ids_flat, char_embs)
    return out.reshape(_B, _L, _EMB)
